# speedup vs baseline: 1.8850x; 1.8850x over previous
import jax
import jax.numpy as jnp
from jax.experimental import pallas as pl

K = 10
ALPHA = 0.1


def _mlp_body(x_ref, w1_ref, b1_ref, w2_ref, b2_ref, o_ref):
    h = jnp.maximum(
        jnp.dot(x_ref[...], w1_ref[...], preferred_element_type=jnp.float32)
        + b1_ref[...], 0.0)
    o_ref[...] = (
        jnp.dot(h, w2_ref[...], preferred_element_type=jnp.float32) + b2_ref[...])


def kernel(x, edge_index, W1, b1, W2, b2):
    n, d = x.shape
    hdim = W1.shape[1]
    c = W2.shape[1]
    blk = 1000
    h = pl.pallas_call(
        _mlp_body,
        grid=(n // blk,),
        in_specs=[
            pl.BlockSpec((blk, d), lambda i: (i, 0)),
            pl.BlockSpec((d, hdim), lambda i: (0, 0)),
            pl.BlockSpec((hdim,), lambda i: (0,)),
            pl.BlockSpec((hdim, c), lambda i: (0, 0)),
            pl.BlockSpec((c,), lambda i: (0,)),
        ],
        out_specs=pl.BlockSpec((blk, c), lambda i: (i, 0)),
        out_shape=jax.ShapeDtypeStruct((n, c), jnp.float32),
    )(x, W1, b1, W2, b2)

    src = edge_index[0]
    dst = edge_index[1]
    ones = jnp.ones(src.shape[0] + n, dtype=h.dtype)
    loop = jnp.arange(n, dtype=edge_index.dtype)
    deg = jax.ops.segment_sum(
        ones, jnp.concatenate([dst, loop]), num_segments=n)
    dinv = jnp.where(deg > 0, deg ** -0.5, 0.0)
    h0 = h
    for _ in range(K):
        hh = h * dinv[:, None]
        agg = jax.ops.segment_sum(hh[src], dst, num_segments=n) + hh
        h = (1.0 - ALPHA) * dinv[:, None] * agg + ALPHA * h0
    return jax.nn.log_softmax(h, axis=1)


# speedup vs baseline: 18.0838x; 9.5937x over previous
import functools

import jax
import jax.numpy as jnp
from jax import lax
from jax.experimental import pallas as pl
from jax.experimental.pallas import tpu as pltpu
from jax.experimental.pallas import tpu_sc as plsc

NP = 10240
CP = 48
DW = 16
NW = 32
NS = 16
CH = 128
RT = NP // NS
BLK = 640
ALPHA = 0.1
K = 10

_MESH = plsc.VectorSubcoreMesh(core_axis_name="c", subcore_axis_name="s")
_SC_PARAMS = pltpu.CompilerParams(use_tc_tiling_on_sc=False)


def _zero_fill(ref, rows, width):
    def body(i, _):
        for j in range(width // 16):
            ref[i, pl.ds(j * 16, 16)] = jnp.zeros((16,), jnp.float32)
        return 0

    lax.fori_loop(0, rows, body, 0)


def _ones_fill(ref, rows, width):
    def body(i, _):
        for j in range(width // 16):
            ref[i, pl.ds(j * 16, 16)] = jnp.ones((16,), jnp.float32)
        return 0

    lax.fori_loop(0, rows, body, 0)


def _deg_body(nch, dst_hbm, degs_hbm, acc, zbuf, ones_v, dst_v):
    c = lax.axis_index("c")
    s = lax.axis_index("s")
    wid = c * NS + s
    _zero_fill(zbuf, RT, DW)
    pltpu.sync_copy(zbuf, acc.at[pl.ds(s * RT, RT)])
    plsc.subcore_barrier()
    _ones_fill(ones_v, CH, DW)
    pltpu.sync_copy(dst_hbm.at[wid], dst_v)

    def body(j, _):
        pltpu.sync_copy(ones_v, acc.at[dst_v.at[j]], add=True)
        return 0

    lax.fori_loop(0, nch, body, 0)
    plsc.subcore_barrier()
    pltpu.sync_copy(acc.at[pl.ds(s * RT, RT)],
                    degs_hbm.at[pl.ds(c * NP + s * RT, RT)])


def _round_body(nch, hh_hbm, src_hbm, dst_hbm, accs_hbm,
                acc, zbuf, src_v, dst_v, rows_v):
    c = lax.axis_index("c")
    s = lax.axis_index("s")
    wid = c * NS + s
    _zero_fill(zbuf, RT, CP)
    pltpu.sync_copy(zbuf, acc.at[pl.ds(s * RT, RT)])
    pltpu.sync_copy(src_hbm.at[wid], src_v)
    pltpu.sync_copy(dst_hbm.at[wid], dst_v)
    plsc.subcore_barrier()

    def body(j, _):
        pltpu.sync_copy(hh_hbm.at[src_v.at[j]], rows_v)
        pltpu.sync_copy(rows_v, acc.at[dst_v.at[j]], add=True)
        return 0

    lax.fori_loop(0, nch, body, 0)
    plsc.subcore_barrier()
    pltpu.sync_copy(acc.at[pl.ds(s * RT, RT)],
                    accs_hbm.at[pl.ds(c * NP + s * RT, RT)])


def _prep_body(x_ref, w1_ref, b1_ref, w2_ref, b2_ref, dg0_ref, dg1_ref,
               h0_ref, hh0_ref, dinv_ref):
    i = pl.program_id(0)
    h = jnp.maximum(
        jnp.dot(x_ref[...], w1_ref[...], preferred_element_type=jnp.float32)
        + b1_ref[...], 0.0)
    h = jnp.dot(h, w2_ref[...], preferred_element_type=jnp.float32) + b2_ref[...]
    rows = i * BLK + lax.broadcasted_iota(jnp.int32, (BLK, 1), 0)
    h = jnp.where(rows < 10000, h, 0.0)
    deg = 1.0 + dg0_ref[:, 0:1] + dg1_ref[:, 0:1]
    dinv = lax.rsqrt(deg)
    h0_ref[...] = h
    hh0_ref[...] = h * dinv
    dinv_ref[...] = dinv


def _update_body(a0_ref, a1_ref, hh_ref, h0_ref, dinv_ref, out_ref):
    dinv = dinv_ref[...]
    hn = (1.0 - ALPHA) * dinv * (a0_ref[...] + a1_ref[...] + hh_ref[...]) \
        + ALPHA * h0_ref[...]
    out_ref[...] = hn * dinv


def _final_body(a0_ref, a1_ref, hh_ref, h0_ref, dinv_ref, out_ref):
    dinv = dinv_ref[...]
    hn = (1.0 - ALPHA) * dinv * (a0_ref[...] + a1_ref[...] + hh_ref[...]) \
        + ALPHA * h0_ref[...]
    l = hn[:, :40]
    m = jnp.max(l, axis=1, keepdims=True)
    e = jnp.exp(l - m)
    out_ref[...] = l - m - jnp.log(jnp.sum(e, axis=1, keepdims=True))


def kernel(x, edge_index, W1, b1, W2, b2):
    n, d = x.shape
    e = edge_index.shape[1]
    hdim = W1.shape[1]
    c0 = W2.shape[1]
    ew = e // NW
    nch = (ew + CH - 1) // CH
    ewp = nch * CH

    x_pad = jnp.zeros((NP, d), x.dtype).at[:n].set(x)
    W2p = jnp.zeros((hdim, CP), W2.dtype).at[:, :c0].set(W2)
    b2p = jnp.zeros((CP,), b2.dtype).at[:c0].set(b2)

    pad = ewp - ew
    pad_src = 10000 + (jnp.arange(pad, dtype=jnp.int32) % 64)
    pad_dst = 10016 + (jnp.arange(pad, dtype=jnp.int32) % 128)
    src3 = jnp.concatenate(
        [edge_index[0].reshape(NW, ew),
         jnp.broadcast_to(pad_src, (NW, pad))], axis=1).reshape(NW, nch, CH)
    dst3 = jnp.concatenate(
        [edge_index[1].reshape(NW, ew),
         jnp.broadcast_to(pad_dst, (NW, pad))], axis=1).reshape(NW, nch, CH)

    deg_call = pl.kernel(
        functools.partial(_deg_body, nch),
        out_type=jax.ShapeDtypeStruct((2 * NP, DW), jnp.float32),
        mesh=_MESH,
        scratch_types=[
            pltpu.VMEM_SHARED((NP, DW), jnp.float32),
            pltpu.VMEM((RT, DW), jnp.float32),
            pltpu.VMEM((CH, DW), jnp.float32),
            pltpu.VMEM((nch, CH), jnp.int32),
        ],
        compiler_params=_SC_PARAMS,
    )
    degs = deg_call(dst3)

    prep_call = pl.pallas_call(
        _prep_body,
        grid=(NP // BLK,),
        in_specs=[
            pl.BlockSpec((BLK, d), lambda i: (i, 0)),
            pl.BlockSpec((d, hdim), lambda i: (0, 0)),
            pl.BlockSpec((hdim,), lambda i: (0,)),
            pl.BlockSpec((hdim, CP), lambda i: (0, 0)),
            pl.BlockSpec((CP,), lambda i: (0,)),
            pl.BlockSpec((BLK, DW), lambda i: (i, 0)),
            pl.BlockSpec((BLK, DW), lambda i: (i + NP // BLK, 0)),
        ],
        out_specs=[
            pl.BlockSpec((BLK, CP), lambda i: (i, 0)),
            pl.BlockSpec((BLK, CP), lambda i: (i, 0)),
            pl.BlockSpec((BLK, 1), lambda i: (i, 0)),
        ],
        out_shape=[
            jax.ShapeDtypeStruct((NP, CP), jnp.float32),
            jax.ShapeDtypeStruct((NP, CP), jnp.float32),
            jax.ShapeDtypeStruct((NP, 1), jnp.float32),
        ],
    )
    h0, hh, dinv = prep_call(x_pad, W1, b1, W2p, b2p, degs, degs)

    round_call = pl.kernel(
        functools.partial(_round_body, nch),
        out_type=jax.ShapeDtypeStruct((2 * NP, CP), jnp.float32),
        mesh=_MESH,
        scratch_types=[
            pltpu.VMEM_SHARED((NP, CP), jnp.float32),
            pltpu.VMEM((RT, CP), jnp.float32),
            pltpu.VMEM((nch, CH), jnp.int32),
            pltpu.VMEM((nch, CH), jnp.int32),
            pltpu.VMEM((CH, CP), jnp.float32),
        ],
        compiler_params=_SC_PARAMS,
    )

    dense_specs = dict(
        grid=(NP // BLK,),
        in_specs=[
            pl.BlockSpec((BLK, CP), lambda i: (i, 0)),
            pl.BlockSpec((BLK, CP), lambda i: (i + NP // BLK, 0)),
            pl.BlockSpec((BLK, CP), lambda i: (i, 0)),
            pl.BlockSpec((BLK, CP), lambda i: (i, 0)),
            pl.BlockSpec((BLK, 1), lambda i: (i, 0)),
        ],
    )
    update_call = pl.pallas_call(
        _update_body,
        out_specs=pl.BlockSpec((BLK, CP), lambda i: (i, 0)),
        out_shape=jax.ShapeDtypeStruct((NP, CP), jnp.float32),
        **dense_specs,
    )
    final_call = pl.pallas_call(
        _final_body,
        out_specs=pl.BlockSpec((BLK, 40), lambda i: (i, 0)),
        out_shape=jax.ShapeDtypeStruct((NP, 40), jnp.float32),
        **dense_specs,
    )

    for k in range(K):
        accs = round_call(hh, src3, dst3)
        if k < K - 1:
            hh = update_call(accs, accs, hh, h0, dinv)
        else:
            out = final_call(accs, accs, hh, h0, dinv)
    return out[:n]


# speedup vs baseline: 24.7702x; 1.3697x over previous
import functools

import jax
import jax.numpy as jnp
from jax import lax
from jax.experimental import pallas as pl
from jax.experimental.pallas import tpu as pltpu
from jax.experimental.pallas import tpu_sc as plsc

NP = 10240
CP = 48
DW = 16
NW = 32
NS = 16
CH = 128
RT = NP // NS
BLK = 640
ALPHA = 0.1
K = 10

_MESH = plsc.VectorSubcoreMesh(core_axis_name="c", subcore_axis_name="s")
_SC_PARAMS = pltpu.CompilerParams(use_tc_tiling_on_sc=False)


def _zero_fill(ref, rows, width):
    def body(i, _):
        for j in range(width // 16):
            ref[i, pl.ds(j * 16, 16)] = jnp.zeros((16,), jnp.float32)
        return 0

    lax.fori_loop(0, rows, body, 0)


def _ones_fill(ref, rows, width):
    def body(i, _):
        for j in range(width // 16):
            ref[i, pl.ds(j * 16, 16)] = jnp.ones((16,), jnp.float32)
        return 0

    lax.fori_loop(0, rows, body, 0)


def _deg_body(nch, dst_hbm, degs_hbm, acc, zbuf, ones_v, dst_v):
    c = lax.axis_index("c")
    s = lax.axis_index("s")
    wid = c * NS + s
    _zero_fill(zbuf, RT, DW)
    pltpu.sync_copy(zbuf, acc.at[pl.ds(s * RT, RT)])
    plsc.subcore_barrier()
    _ones_fill(ones_v, CH, DW)
    pltpu.sync_copy(dst_hbm.at[wid], dst_v)

    def body(j, _):
        pltpu.sync_copy(ones_v, acc.at[dst_v.at[j]], add=True)
        return 0

    lax.fori_loop(0, nch, body, 0)
    plsc.subcore_barrier()
    pltpu.sync_copy(acc.at[pl.ds(s * RT, RT)],
                    degs_hbm.at[pl.ds(c * NP + s * RT, RT)])


def _round_body(nch, hh_hbm, src_hbm, dst_hbm, accs_hbm,
                acc, zbuf, src_v, dst_v, buf0, buf1, sem0, sem1):
    c = lax.axis_index("c")
    s = lax.axis_index("s")
    wid = c * NS + s
    _zero_fill(zbuf, RT, CP)
    pltpu.sync_copy(zbuf, acc.at[pl.ds(s * RT, RT)])
    pltpu.sync_copy(src_hbm.at[wid], src_v)
    pltpu.sync_copy(dst_hbm.at[wid], dst_v)
    plsc.subcore_barrier()

    pltpu.async_copy(hh_hbm.at[src_v.at[0]], buf0, sem0)

    def body(j2, _):
        base = j2 * 2
        pltpu.async_copy(hh_hbm.at[src_v.at[base + 1]], buf1, sem1)
        pltpu.make_async_copy(hh_hbm.at[src_v.at[base]], buf0, sem0).wait()
        pltpu.sync_copy(buf0, acc.at[dst_v.at[base]], add=True)
        pltpu.async_copy(hh_hbm.at[src_v.at[base + 2]], buf0, sem0)
        pltpu.make_async_copy(hh_hbm.at[src_v.at[base + 1]], buf1, sem1).wait()
        pltpu.sync_copy(buf1, acc.at[dst_v.at[base + 1]], add=True)
        return 0

    lax.fori_loop(0, nch // 2, body, 0)
    pltpu.make_async_copy(hh_hbm.at[src_v.at[nch]], buf0, sem0).wait()
    plsc.subcore_barrier()
    pltpu.sync_copy(acc.at[pl.ds(s * RT, RT)],
                    accs_hbm.at[pl.ds(c * NP + s * RT, RT)])


def _prep_body(x_ref, w1_ref, b1_ref, w2_ref, b2_ref, dg0_ref, dg1_ref,
               h0_ref, hh0_ref, dinv_ref):
    i = pl.program_id(0)
    h = jnp.maximum(
        jnp.dot(x_ref[...], w1_ref[...], preferred_element_type=jnp.float32)
        + b1_ref[...], 0.0)
    h = jnp.dot(h, w2_ref[...], preferred_element_type=jnp.float32) + b2_ref[...]
    rows = i * BLK + lax.broadcasted_iota(jnp.int32, (BLK, 1), 0)
    h = jnp.where(rows < 10000, h, 0.0)
    deg = 1.0 + dg0_ref[:, 0:1] + dg1_ref[:, 0:1]
    dinv = lax.rsqrt(deg)
    h0_ref[...] = h
    hh0_ref[...] = h * dinv
    dinv_ref[...] = dinv


def _update_body(a0_ref, a1_ref, hh_ref, h0_ref, dinv_ref, out_ref):
    dinv = dinv_ref[...]
    hn = (1.0 - ALPHA) * dinv * (a0_ref[...] + a1_ref[...] + hh_ref[...]) \
        + ALPHA * h0_ref[...]
    out_ref[...] = hn * dinv


def _final_body(a0_ref, a1_ref, hh_ref, h0_ref, dinv_ref, out_ref):
    dinv = dinv_ref[...]
    hn = (1.0 - ALPHA) * dinv * (a0_ref[...] + a1_ref[...] + hh_ref[...]) \
        + ALPHA * h0_ref[...]
    l = hn[:, :40]
    m = jnp.max(l, axis=1, keepdims=True)
    e = jnp.exp(l - m)
    out_ref[...] = l - m - jnp.log(jnp.sum(e, axis=1, keepdims=True))


def kernel(x, edge_index, W1, b1, W2, b2):
    n, d = x.shape
    e = edge_index.shape[1]
    hdim = W1.shape[1]
    c0 = W2.shape[1]
    ew = e // NW
    nch = -(-ew // CH)
    nch += nch % 2
    ewp = nch * CH

    x_pad = jnp.zeros((NP, d), x.dtype).at[:n].set(x)
    W2p = jnp.zeros((hdim, CP), W2.dtype).at[:, :c0].set(W2)
    b2p = jnp.zeros((CP,), b2.dtype).at[:c0].set(b2)

    pad = ewp - ew
    pad_s = ewp + 2 * CH - ew
    pad_src = 10000 + (jnp.arange(pad_s, dtype=jnp.int32) % 64)
    pad_dst = 10016 + (jnp.arange(pad, dtype=jnp.int32) % 128)
    src3 = jnp.concatenate(
        [edge_index[0].reshape(NW, ew),
         jnp.broadcast_to(pad_src, (NW, pad_s))], axis=1).reshape(NW, nch + 2, CH)
    dst3 = jnp.concatenate(
        [edge_index[1].reshape(NW, ew),
         jnp.broadcast_to(pad_dst, (NW, pad))], axis=1).reshape(NW, nch, CH)

    deg_call = pl.kernel(
        functools.partial(_deg_body, nch),
        out_type=jax.ShapeDtypeStruct((2 * NP, DW), jnp.float32),
        mesh=_MESH,
        scratch_types=[
            pltpu.VMEM_SHARED((NP, DW), jnp.float32),
            pltpu.VMEM((RT, DW), jnp.float32),
            pltpu.VMEM((CH, DW), jnp.float32),
            pltpu.VMEM((nch, CH), jnp.int32),
        ],
        compiler_params=_SC_PARAMS,
    )
    degs = deg_call(dst3)

    prep_call = pl.pallas_call(
        _prep_body,
        grid=(NP // BLK,),
        in_specs=[
            pl.BlockSpec((BLK, d), lambda i: (i, 0)),
            pl.BlockSpec((d, hdim), lambda i: (0, 0)),
            pl.BlockSpec((hdim,), lambda i: (0,)),
            pl.BlockSpec((hdim, CP), lambda i: (0, 0)),
            pl.BlockSpec((CP,), lambda i: (0,)),
            pl.BlockSpec((BLK, DW), lambda i: (i, 0)),
            pl.BlockSpec((BLK, DW), lambda i: (i + NP // BLK, 0)),
        ],
        out_specs=[
            pl.BlockSpec((BLK, CP), lambda i: (i, 0)),
            pl.BlockSpec((BLK, CP), lambda i: (i, 0)),
            pl.BlockSpec((BLK, 1), lambda i: (i, 0)),
        ],
        out_shape=[
            jax.ShapeDtypeStruct((NP, CP), jnp.float32),
            jax.ShapeDtypeStruct((NP, CP), jnp.float32),
            jax.ShapeDtypeStruct((NP, 1), jnp.float32),
        ],
    )
    h0, hh, dinv = prep_call(x_pad, W1, b1, W2p, b2p, degs, degs)

    round_call = pl.kernel(
        functools.partial(_round_body, nch),
        out_type=jax.ShapeDtypeStruct((2 * NP, CP), jnp.float32),
        mesh=_MESH,
        scratch_types=[
            pltpu.VMEM_SHARED((NP, CP), jnp.float32),
            pltpu.VMEM((RT, CP), jnp.float32),
            pltpu.VMEM((nch + 2, CH), jnp.int32),
            pltpu.VMEM((nch, CH), jnp.int32),
            pltpu.VMEM((CH, CP), jnp.float32),
            pltpu.VMEM((CH, CP), jnp.float32),
            pltpu.SemaphoreType.DMA,
            pltpu.SemaphoreType.DMA,
        ],
        compiler_params=_SC_PARAMS,
    )

    dense_specs = dict(
        grid=(NP // BLK,),
        in_specs=[
            pl.BlockSpec((BLK, CP), lambda i: (i, 0)),
            pl.BlockSpec((BLK, CP), lambda i: (i + NP // BLK, 0)),
            pl.BlockSpec((BLK, CP), lambda i: (i, 0)),
            pl.BlockSpec((BLK, CP), lambda i: (i, 0)),
            pl.BlockSpec((BLK, 1), lambda i: (i, 0)),
        ],
    )
    update_call = pl.pallas_call(
        _update_body,
        out_specs=pl.BlockSpec((BLK, CP), lambda i: (i, 0)),
        out_shape=jax.ShapeDtypeStruct((NP, CP), jnp.float32),
        **dense_specs,
    )
    final_call = pl.pallas_call(
        _final_body,
        out_specs=pl.BlockSpec((BLK, 40), lambda i: (i, 0)),
        out_shape=jax.ShapeDtypeStruct((NP, 40), jnp.float32),
        **dense_specs,
    )

    for k in range(K):
        accs = round_call(hh, src3, dst3)
        if k < K - 1:
            hh = update_call(accs, accs, hh, h0, dinv)
        else:
            out = final_call(accs, accs, hh, h0, dinv)
    return out[:n]
